# baseline (speedup 1.0000x reference)
import functools
import math

import jax
import jax.numpy as jnp
import numpy as np
from jax import lax
from jax.experimental import pallas as pl
from jax.experimental.pallas import tpu as pltpu
from jax.experimental.pallas import tpu_sc as plsc

SS = ((64, 64), (32, 32), (16, 16), (8, 8))
BS = 4
LQ = 5440
D = 256
NH = 8
NL = 4
NP = 4
HD = D // NH
TP = NH * NL * NP
LV = sum(h * w for h, w in SS)
TQ = 320
NT = LQ // TQ
CH = 160
NCH = LQ // CH
NK = 4 * NL * NP

_colW = np.zeros((128,), np.float32)
_colH = np.zeros((128,), np.float32)
_colS = np.zeros((128,), np.float32)
_starts = [0, 4096, 5120, 5376]
for _h in range(NH):
    for _l in range(NL):
        for _p in range(NP):
            c = _h * 16 + _l * 4 + _p
            _colW[c] = SS[_l][1]
            _colH[c] = SS[_l][0]
            _colS[c] = _starts[_l]
_G = np.zeros((128, 128), np.float32)
for c in range(128):
    g = c // 16
    _G[c, g * 16:(g + 1) * 16] = 1.0
_Ex = np.zeros((8, 128), np.float32)
_Ey = np.zeros((8, 128), np.float32)
for c in range(128):
    _l = (c % 16) // 4
    _Ex[2 * _l, c] = 1.0
    _Ey[2 * _l + 1, c] = 1.0

_CORNERS = ((0.0, 0.0), (0.0, 1.0), (1.0, 0.0), (1.0, 1.0))
_HI = lax.Precision.HIGHEST


def _prep_body(value_b, query_b, ref_b, WvT, bv, WoxT, box, WoyT, boy,
               WaT, ba, G, Ex, Ey, colW, colH, colS,
               v_out, idx_out, cw_out):
    value_b = value_b[0]
    query_b = query_b[0]
    ref_b = ref_b[0]
    v = jnp.dot(value_b, WvT[...], preferred_element_type=jnp.float32,
                precision=_HI) + bv[...]
    v_out[0] = v
    offx = jnp.dot(query_b, WoxT[...], preferred_element_type=jnp.float32,
                   precision=_HI) + box[...]
    offy = jnp.dot(query_b, WoyT[...], preferred_element_type=jnp.float32,
                   precision=_HI) + boy[...]
    logits = jnp.dot(query_b, WaT[...], preferred_element_type=jnp.float32,
                     precision=_HI) + ba[...]
    m = jnp.max(logits, axis=-1, keepdims=True)
    e = jnp.exp(logits - m)
    denom = jnp.dot(e, G[...], preferred_element_type=jnp.float32,
                    precision=_HI)
    aw = e / denom
    refx = jnp.dot(ref_b, Ex[...], preferred_element_type=jnp.float32,
                   precision=_HI)
    refy = jnp.dot(ref_b, Ey[...], preferred_element_type=jnp.float32,
                   precision=_HI)
    cW = colW[...]
    cH = colH[...]
    cS = colS[...]
    px = refx * cW + offx - 0.5
    py = refy * cH + offy - 0.5
    x0 = jnp.floor(px)
    y0 = jnp.floor(py)
    fx = px - x0
    fy = py - y0
    for ci, (dx, dy) in enumerate(_CORNERS):
        xi = x0 + dx
        yi = y0 + dy
        valid = ((xi >= 0.0) & (xi <= cW - 1.0)
                 & (yi >= 0.0) & (yi <= cH - 1.0))
        wx = fx if dx else 1.0 - fx
        wy = fy if dy else 1.0 - fy
        cw = jnp.where(valid, aw * wx * wy, 0.0)
        xc = jnp.clip(xi, 0.0, cW - 1.0)
        yc = jnp.clip(yi, 0.0, cH - 1.0)
        rows = (cS + yc * cW + xc).astype(jnp.int32)
        idx_out[0, :, ci, :] = rows * 17
        cw_out[0, :, ci, :] = cw


def _outproj_body(x_b, WoT, bo, out_b):
    out_b[0] = (jnp.dot(x_b[0], WoT[...], preferred_element_type=jnp.float32,
                        precision=_HI) + bo[...])


_sc_mesh = plsc.VectorSubcoreMesh(core_axis_name="c", subcore_axis_name="s")


@functools.partial(
    pl.kernel,
    out_type=jax.ShapeDtypeStruct((BS * NH * LQ * 33,), jnp.float32),
    mesh=_sc_mesh,
    compiler_params=pltpu.CompilerParams(needs_layout_passes=False),
    scratch_types=[
        pltpu.VMEM((LV * 17,), jnp.int32),
        pltpu.VMEM((CH * 65,), jnp.int32),
        pltpu.VMEM((CH * 65,), jnp.float32),
        pltpu.VMEM((CH * 33,), jnp.float32),
    ],
)
def _sc_sample(tbl_hbm, idx_hbm, cw_hbm, out_hbm, tbl_v, idx_v, cw_v, out_v):
    cid = lax.axis_index("c")
    sid = lax.axis_index("s")
    wid = sid * 2 + cid
    pltpu.sync_copy(tbl_hbm.at[pl.ds(wid * (LV * 17), LV * 17)], tbl_v)
    lanes = lax.iota(jnp.int32, 16)

    def chunk_body(ci, _):
        q0 = ci * CH
        off = wid * (LQ * 65) + q0 * 65
        pltpu.sync_copy(idx_hbm.at[pl.ds(off, CH * 65)], idx_v)
        pltpu.sync_copy(cw_hbm.at[pl.ds(off, CH * 65)], cw_v)

        def qblk_body(qi, _):
            qrow = qi * 16 + lanes
            qk = qrow * 65
            qd = qrow * 33
            zero = jnp.zeros((16,), jnp.float32)
            accs = (zero,) * HD

            @plsc.parallel_loop(0, NK, carry=accs, unroll=2)
            def accs(k, accs):
                base = qk + k
                rowb = plsc.load_gather(idx_v, [base])
                w = plsc.load_gather(cw_v, [base])
                accs = list(accs)
                for j in range(16):
                    g = plsc.load_gather(tbl_v, [rowb + j])
                    gb = plsc.bitcast(g, jnp.bfloat16)
                    ea, eb = plsc.unpack(
                        gb, format=plsc.PackFormat.INTERLEAVED,
                        preferred_element_type=jnp.float32)
                    accs[2 * j] = accs[2 * j] + ea * w
                    accs[2 * j + 1] = accs[2 * j + 1] + eb * w
                return tuple(accs)
            for d in range(HD):
                plsc.store_scatter(out_v, [qd + d], accs[d])
            return 0

        lax.fori_loop(0, CH // 16, qblk_body, 0)
        pltpu.sync_copy(out_v, out_hbm.at[pl.ds(wid * (LQ * 33) + q0 * 33, CH * 33)])
        return 0

    lax.fori_loop(0, NCH, chunk_body, 0)


@jax.jit
def _run(query, reference_points, value, W_value, b_value, W_off, b_off,
         W_attn, b_attn, W_out, b_out):
    WvT = W_value.T
    WoxT = W_off[0::2].T
    WoyT = W_off[1::2].T
    box = b_off[0::2]
    boy = b_off[1::2]
    WaT = W_attn.T
    ref_flat = reference_points.reshape(BS, LQ, NL * 2)

    tok = lambda i, b: (b, i, 0)
    full = lambda i, b: (0, 0)
    vec = lambda i, b: (0,)
    v_p, idx_p, cw_p = pl.pallas_call(
        _prep_body,
        grid=(NT, BS),
        in_specs=[
            pl.BlockSpec((1, TQ, D), tok),
            pl.BlockSpec((1, TQ, D), tok),
            pl.BlockSpec((1, TQ, NL * 2), tok),
            pl.BlockSpec((D, D), full),
            pl.BlockSpec((D,), vec),
            pl.BlockSpec((D, 128), full),
            pl.BlockSpec((128,), vec),
            pl.BlockSpec((D, 128), full),
            pl.BlockSpec((128,), vec),
            pl.BlockSpec((D, 128), full),
            pl.BlockSpec((128,), vec),
            pl.BlockSpec((128, 128), full),
            pl.BlockSpec((8, 128), full),
            pl.BlockSpec((8, 128), full),
            pl.BlockSpec((128,), vec),
            pl.BlockSpec((128,), vec),
            pl.BlockSpec((128,), vec),
        ],
        out_specs=[
            pl.BlockSpec((1, TQ, D), tok),
            pl.BlockSpec((1, TQ, 4, 128), lambda i, b: (b, i, 0, 0)),
            pl.BlockSpec((1, TQ, 4, 128), lambda i, b: (b, i, 0, 0)),
        ],
        out_shape=[
            jax.ShapeDtypeStruct((BS, LQ, D), jnp.float32),
            jax.ShapeDtypeStruct((BS, LQ, 4, 128), jnp.int32),
            jax.ShapeDtypeStruct((BS, LQ, 4, 128), jnp.float32),
        ],
    )(value, query, ref_flat, WvT, b_value, WoxT, box, WoyT, boy,
      WaT, b_attn, jnp.asarray(_G), jnp.asarray(_Ex), jnp.asarray(_Ey),
      jnp.asarray(_colW), jnp.asarray(_colH), jnp.asarray(_colS))

    v4 = v_p.reshape(BS, LV, NH, HD).transpose(0, 2, 1, 3)
    vb = v4.astype(jnp.bfloat16).reshape(BS, NH, LV, 16, 2)
    tbl = lax.bitcast_convert_type(vb, jnp.int32)
    tbl = jnp.pad(tbl, ((0, 0), (0, 0), (0, 0), (0, 1)))
    tbl = tbl.reshape(BS * NH * LV * 17)
    idx_r = idx_p.reshape(BS, LQ, 4, NH, 16).transpose(0, 3, 1, 2, 4)
    idx_r = idx_r.reshape(BS, NH, LQ, NK)
    idx_r = jnp.pad(idx_r, ((0, 0), (0, 0), (0, 0), (0, 1)))
    idx_r = idx_r.reshape(BS * NH * LQ * 65)
    cw_r = cw_p.reshape(BS, LQ, 4, NH, 16).transpose(0, 3, 1, 2, 4)
    cw_r = cw_r.reshape(BS, NH, LQ, NK)
    cw_r = jnp.pad(cw_r, ((0, 0), (0, 0), (0, 0), (0, 1)))
    cw_r = cw_r.reshape(BS * NH * LQ * 65)

    samp = _sc_sample(tbl, idx_r, cw_r)
    samp = samp.reshape(BS, NH, LQ, 33)[..., :HD]
    samp = samp.transpose(0, 2, 1, 3).reshape(BS, LQ, D)

    out = pl.pallas_call(
        _outproj_body,
        grid=(NT, BS),
        in_specs=[
            pl.BlockSpec((1, TQ, D), tok),
            pl.BlockSpec((D, D), full),
            pl.BlockSpec((D,), vec),
        ],
        out_specs=pl.BlockSpec((1, TQ, D), tok),
        out_shape=jax.ShapeDtypeStruct((BS, LQ, D), jnp.float32),
    )(samp, W_out.T, b_out)
    return out


def kernel(query, reference_points, value, value_spatial_shapes,
           value_level_start_index, W_value, b_value, W_off, b_off,
           W_attn, b_attn, W_out, b_out):
    return _run(query, reference_points, value, W_value, b_value,
                W_off, b_off, W_attn, b_attn, W_out, b_out)


# speedup vs baseline: 1.0970x; 1.0970x over previous
import functools
import math

import jax
import jax.numpy as jnp
import numpy as np
from jax import lax
from jax.experimental import pallas as pl
from jax.experimental.pallas import tpu as pltpu
from jax.experimental.pallas import tpu_sc as plsc

SS = ((64, 64), (32, 32), (16, 16), (8, 8))
BS = 4
LQ = 5440
D = 256
NH = 8
NL = 4
NP = 4
HD = D // NH
TP = NH * NL * NP
LV = sum(h * w for h, w in SS)
TQ = 320
NT = LQ // TQ
CH = 160
NCH = LQ // CH
NK = 4 * NL * NP

_colW = np.zeros((128,), np.float32)
_colH = np.zeros((128,), np.float32)
_colS = np.zeros((128,), np.float32)
_starts = [0, 4096, 5120, 5376]
for _h in range(NH):
    for _l in range(NL):
        for _p in range(NP):
            c = _h * 16 + _l * 4 + _p
            _colW[c] = SS[_l][1]
            _colH[c] = SS[_l][0]
            _colS[c] = _starts[_l]
_G = np.zeros((128, 128), np.float32)
for c in range(128):
    g = c // 16
    _G[c, g * 16:(g + 1) * 16] = 1.0
_Ex = np.zeros((8, 128), np.float32)
_Ey = np.zeros((8, 128), np.float32)
for c in range(128):
    _l = (c % 16) // 4
    _Ex[2 * _l, c] = 1.0
    _Ey[2 * _l + 1, c] = 1.0

_CORNERS = ((0.0, 0.0), (0.0, 1.0), (1.0, 0.0), (1.0, 1.0))
_HI = lax.Precision.HIGHEST


def _prep_body(value_b, query_b, ref_b, WvT, bv, WoxT, box, WoyT, boy,
               WaT, ba, G, Ex, Ey, colW, colH, colS,
               v_out, idx_out, cw_out):
    value_b = value_b[0]
    query_b = query_b[0]
    ref_b = ref_b[0]
    v = jnp.dot(value_b, WvT[...], preferred_element_type=jnp.float32,
                precision=_HI) + bv[...]
    v_out[0] = v
    offx = jnp.dot(query_b, WoxT[...], preferred_element_type=jnp.float32,
                   precision=_HI) + box[...]
    offy = jnp.dot(query_b, WoyT[...], preferred_element_type=jnp.float32,
                   precision=_HI) + boy[...]
    logits = jnp.dot(query_b, WaT[...], preferred_element_type=jnp.float32,
                     precision=_HI) + ba[...]
    m = jnp.max(logits, axis=-1, keepdims=True)
    e = jnp.exp(logits - m)
    denom = jnp.dot(e, G[...], preferred_element_type=jnp.float32,
                    precision=_HI)
    aw = e / denom
    refx = jnp.dot(ref_b, Ex[...], preferred_element_type=jnp.float32,
                   precision=_HI)
    refy = jnp.dot(ref_b, Ey[...], preferred_element_type=jnp.float32,
                   precision=_HI)
    cW = colW[...]
    cH = colH[...]
    cS = colS[...]
    px = refx * cW + offx - 0.5
    py = refy * cH + offy - 0.5
    x0 = jnp.floor(px)
    y0 = jnp.floor(py)
    fx = px - x0
    fy = py - y0
    for ci, (dx, dy) in enumerate(_CORNERS):
        xi = x0 + dx
        yi = y0 + dy
        valid = ((xi >= 0.0) & (xi <= cW - 1.0)
                 & (yi >= 0.0) & (yi <= cH - 1.0))
        wx = fx if dx else 1.0 - fx
        wy = fy if dy else 1.0 - fy
        cw = jnp.where(valid, aw * wx * wy, 0.0)
        xc = jnp.clip(xi, 0.0, cW - 1.0)
        yc = jnp.clip(yi, 0.0, cH - 1.0)
        rows = (cS + yc * cW + xc).astype(jnp.int32)
        idx_out[0, :, ci, :] = rows * 17
        cw_out[0, :, ci, :] = cw


def _outproj_body(x_b, WoT, bo, out_b):
    out_b[0] = (jnp.dot(x_b[0], WoT[...], preferred_element_type=jnp.float32,
                        precision=_HI) + bo[...])


_sc_mesh = plsc.VectorSubcoreMesh(core_axis_name="c", subcore_axis_name="s")


@functools.partial(
    pl.kernel,
    out_type=jax.ShapeDtypeStruct((BS * NH * LQ * 33,), jnp.float32),
    mesh=_sc_mesh,
    compiler_params=pltpu.CompilerParams(needs_layout_passes=False),
    scratch_types=[
        pltpu.VMEM((LV * 17,), jnp.int32),
        pltpu.VMEM((CH * 65,), jnp.int32),
        pltpu.VMEM((CH * 65,), jnp.float32),
        pltpu.VMEM((CH * 33,), jnp.float32),
    ],
)
def _sc_sample(tbl_hbm, idx_hbm, cw_hbm, out_hbm, tbl_v, idx_v, cw_v, out_v):
    cid = lax.axis_index("c")
    sid = lax.axis_index("s")
    wid = sid * 2 + cid
    pltpu.sync_copy(tbl_hbm.at[pl.ds(wid * (LV * 17), LV * 17)], tbl_v)
    lanes = lax.iota(jnp.int32, 16)

    def chunk_body(ci, _):
        q0 = ci * CH
        off = wid * (LQ * 65) + q0 * 65
        pltpu.sync_copy(idx_hbm.at[pl.ds(off, CH * 65)], idx_v)
        pltpu.sync_copy(cw_hbm.at[pl.ds(off, CH * 65)], cw_v)

        def qblk_body(qi, _):
            qrow = qi * 16 + lanes
            qk = qrow * 65
            qd = qrow * 33
            zero = jnp.zeros((16,), jnp.float32)
            accs = (zero,) * HD

            def k_body(k, accs):
                base = qk + k
                rowb = plsc.load_gather(idx_v, [base])
                w = plsc.load_gather(cw_v, [base])
                accs = list(accs)
                for j in range(16):
                    g = plsc.load_gather(tbl_v, [rowb + j])
                    gb = plsc.bitcast(g, jnp.bfloat16)
                    ea, eb = plsc.unpack(
                        gb, format=plsc.PackFormat.INTERLEAVED,
                        preferred_element_type=jnp.float32)
                    accs[2 * j] = accs[2 * j] + ea * w
                    accs[2 * j + 1] = accs[2 * j + 1] + eb * w
                return tuple(accs)

            accs = lax.fori_loop(0, NK, k_body, accs)
            for d in range(HD):
                plsc.store_scatter(out_v, [qd + d], accs[d])
            return 0

        lax.fori_loop(0, CH // 16, qblk_body, 0)
        pltpu.sync_copy(out_v, out_hbm.at[pl.ds(wid * (LQ * 33) + q0 * 33, CH * 33)])
        return 0

    lax.fori_loop(0, NCH, chunk_body, 0)


@jax.jit
def _run(query, reference_points, value, W_value, b_value, W_off, b_off,
         W_attn, b_attn, W_out, b_out):
    WvT = W_value.T
    WoxT = W_off[0::2].T
    WoyT = W_off[1::2].T
    box = b_off[0::2]
    boy = b_off[1::2]
    WaT = W_attn.T
    ref_flat = reference_points.reshape(BS, LQ, NL * 2)

    tok = lambda i, b: (b, i, 0)
    full = lambda i, b: (0, 0)
    vec = lambda i, b: (0,)
    v_p, idx_p, cw_p = pl.pallas_call(
        _prep_body,
        grid=(NT, BS),
        in_specs=[
            pl.BlockSpec((1, TQ, D), tok),
            pl.BlockSpec((1, TQ, D), tok),
            pl.BlockSpec((1, TQ, NL * 2), tok),
            pl.BlockSpec((D, D), full),
            pl.BlockSpec((D,), vec),
            pl.BlockSpec((D, 128), full),
            pl.BlockSpec((128,), vec),
            pl.BlockSpec((D, 128), full),
            pl.BlockSpec((128,), vec),
            pl.BlockSpec((D, 128), full),
            pl.BlockSpec((128,), vec),
            pl.BlockSpec((128, 128), full),
            pl.BlockSpec((8, 128), full),
            pl.BlockSpec((8, 128), full),
            pl.BlockSpec((128,), vec),
            pl.BlockSpec((128,), vec),
            pl.BlockSpec((128,), vec),
        ],
        out_specs=[
            pl.BlockSpec((1, TQ, D), tok),
            pl.BlockSpec((1, TQ, 4, 128), lambda i, b: (b, i, 0, 0)),
            pl.BlockSpec((1, TQ, 4, 128), lambda i, b: (b, i, 0, 0)),
        ],
        out_shape=[
            jax.ShapeDtypeStruct((BS, LQ, D), jnp.float32),
            jax.ShapeDtypeStruct((BS, LQ, 4, 128), jnp.int32),
            jax.ShapeDtypeStruct((BS, LQ, 4, 128), jnp.float32),
        ],
    )(value, query, ref_flat, WvT, b_value, WoxT, box, WoyT, boy,
      WaT, b_attn, jnp.asarray(_G), jnp.asarray(_Ex), jnp.asarray(_Ey),
      jnp.asarray(_colW), jnp.asarray(_colH), jnp.asarray(_colS))

    v4 = v_p.reshape(BS, LV, NH, HD).transpose(0, 2, 1, 3)
    vb = v4.astype(jnp.bfloat16).reshape(BS, NH, LV, 16, 2)
    tbl = lax.bitcast_convert_type(vb, jnp.int32)
    tbl = jnp.pad(tbl, ((0, 0), (0, 0), (0, 0), (0, 1)))
    tbl = tbl.reshape(BS * NH * LV * 17)
    idx_r = idx_p.reshape(BS, LQ, 4, NH, 16).transpose(0, 3, 1, 2, 4)
    idx_r = idx_r.reshape(BS, NH, LQ, NK)
    idx_r = jnp.pad(idx_r, ((0, 0), (0, 0), (0, 0), (0, 1)))
    idx_r = idx_r.reshape(BS * NH * LQ * 65)
    cw_r = cw_p.reshape(BS, LQ, 4, NH, 16).transpose(0, 3, 1, 2, 4)
    cw_r = cw_r.reshape(BS, NH, LQ, NK)
    cw_r = jnp.pad(cw_r, ((0, 0), (0, 0), (0, 0), (0, 1)))
    cw_r = cw_r.reshape(BS * NH * LQ * 65)

    samp = _sc_sample(tbl, idx_r, cw_r)
    samp = samp.reshape(BS, NH, LQ, 33)[..., :HD]
    samp = samp.transpose(0, 2, 1, 3).reshape(BS, LQ, D)

    out = pl.pallas_call(
        _outproj_body,
        grid=(NT, BS),
        in_specs=[
            pl.BlockSpec((1, TQ, D), tok),
            pl.BlockSpec((D, D), full),
            pl.BlockSpec((D,), vec),
        ],
        out_specs=pl.BlockSpec((1, TQ, D), tok),
        out_shape=jax.ShapeDtypeStruct((BS, LQ, D), jnp.float32),
    )(samp, W_out.T, b_out)
    return out


def kernel(query, reference_points, value, value_spatial_shapes,
           value_level_start_index, W_value, b_value, W_off, b_off,
           W_attn, b_attn, W_out, b_out):
    return _run(query, reference_points, value, W_value, b_value,
                W_off, b_off, W_attn, b_attn, W_out, b_out)


# speedup vs baseline: 1.1167x; 1.0180x over previous
import functools
import math

import jax
import jax.numpy as jnp
import numpy as np
from jax import lax
from jax.experimental import pallas as pl
from jax.experimental.pallas import tpu as pltpu
from jax.experimental.pallas import tpu_sc as plsc

SS = ((64, 64), (32, 32), (16, 16), (8, 8))
BS = 4
LQ = 5440
D = 256
NH = 8
NL = 4
NP = 4
HD = D // NH
TP = NH * NL * NP
LV = sum(h * w for h, w in SS)
TQ = 320
NT = LQ // TQ
CH = 160
NCH = LQ // CH
NK = 4 * NL * NP

_colW = np.zeros((128,), np.float32)
_colH = np.zeros((128,), np.float32)
_colS = np.zeros((128,), np.float32)
_starts = [0, 4096, 5120, 5376]
for _h in range(NH):
    for _l in range(NL):
        for _p in range(NP):
            c = _h * 16 + _l * 4 + _p
            _colW[c] = SS[_l][1]
            _colH[c] = SS[_l][0]
            _colS[c] = _starts[_l]
_G = np.zeros((128, 128), np.float32)
for c in range(128):
    g = c // 16
    _G[c, g * 16:(g + 1) * 16] = 1.0
_Ex = np.zeros((8, 128), np.float32)
_Ey = np.zeros((8, 128), np.float32)
for c in range(128):
    _l = (c % 16) // 4
    _Ex[2 * _l, c] = 1.0
    _Ey[2 * _l + 1, c] = 1.0

_CORNERS = ((0.0, 0.0), (0.0, 1.0), (1.0, 0.0), (1.0, 1.0))
_HI = lax.Precision.HIGHEST


def _prep_body(value_b, query_b, ref_b, WvT, bv, WoxT, box, WoyT, boy,
               WaT, ba, G, Ex, Ey, colW, colH, colS,
               v_out, idx_out, cw_out):
    value_b = value_b[0]
    query_b = query_b[0]
    ref_b = ref_b[0]
    v = jnp.dot(value_b, WvT[...], preferred_element_type=jnp.float32,
                precision=lax.Precision.DEFAULT) + bv[...]
    v_out[0] = v
    offx = jnp.dot(query_b, WoxT[...], preferred_element_type=jnp.float32,
                   precision=_HI) + box[...]
    offy = jnp.dot(query_b, WoyT[...], preferred_element_type=jnp.float32,
                   precision=_HI) + boy[...]
    logits = jnp.dot(query_b, WaT[...], preferred_element_type=jnp.float32,
                     precision=_HI) + ba[...]
    m = jnp.max(logits, axis=-1, keepdims=True)
    e = jnp.exp(logits - m)
    denom = jnp.dot(e, G[...], preferred_element_type=jnp.float32,
                    precision=_HI)
    aw = e / denom
    refx = jnp.dot(ref_b, Ex[...], preferred_element_type=jnp.float32,
                   precision=_HI)
    refy = jnp.dot(ref_b, Ey[...], preferred_element_type=jnp.float32,
                   precision=_HI)
    cW = colW[...]
    cH = colH[...]
    cS = colS[...]
    px = refx * cW + offx - 0.5
    py = refy * cH + offy - 0.5
    x0 = jnp.floor(px)
    y0 = jnp.floor(py)
    fx = px - x0
    fy = py - y0
    for ci, (dx, dy) in enumerate(_CORNERS):
        xi = x0 + dx
        yi = y0 + dy
        valid = ((xi >= 0.0) & (xi <= cW - 1.0)
                 & (yi >= 0.0) & (yi <= cH - 1.0))
        wx = fx if dx else 1.0 - fx
        wy = fy if dy else 1.0 - fy
        cw = jnp.where(valid, aw * wx * wy, 0.0)
        xc = jnp.clip(xi, 0.0, cW - 1.0)
        yc = jnp.clip(yi, 0.0, cH - 1.0)
        rows = (cS + yc * cW + xc).astype(jnp.int32)
        idx_out[0, :, ci, :] = rows * 17
        cw_out[0, :, ci, :] = cw


def _outproj_body(x_b, WoT, bo, out_b):
    acc = bo[...]
    for hh in range(NH):
        acc = acc + jnp.dot(x_b[0, hh, :, :HD],
                            WoT[pl.ds(hh * HD, HD), :],
                            preferred_element_type=jnp.float32,
                            precision=_HI)
    out_b[0] = acc


_sc_mesh = plsc.VectorSubcoreMesh(core_axis_name="c", subcore_axis_name="s")


@functools.partial(
    pl.kernel,
    out_type=jax.ShapeDtypeStruct((BS * NH * LQ * 33,), jnp.float32),
    mesh=_sc_mesh,
    compiler_params=pltpu.CompilerParams(needs_layout_passes=False),
    scratch_types=[
        pltpu.VMEM((LV * 17,), jnp.int32),
        pltpu.VMEM((CH * 65,), jnp.int32),
        pltpu.VMEM((CH * 65,), jnp.float32),
        pltpu.VMEM((CH * 33,), jnp.float32),
    ],
)
def _sc_sample(tbl_hbm, idx_hbm, cw_hbm, out_hbm, tbl_v, idx_v, cw_v, out_v):
    cid = lax.axis_index("c")
    sid = lax.axis_index("s")
    wid = sid * 2 + cid
    pltpu.sync_copy(tbl_hbm.at[pl.ds(wid * (LV * 17), LV * 17)], tbl_v)
    lanes = lax.iota(jnp.int32, 16)

    def chunk_body(ci, _):
        q0 = ci * CH
        off = wid * (LQ * 65) + q0 * 65
        pltpu.sync_copy(idx_hbm.at[pl.ds(off, CH * 65)], idx_v)
        pltpu.sync_copy(cw_hbm.at[pl.ds(off, CH * 65)], cw_v)

        def qblk_body(qi, _):
            qrow = qi * 16 + lanes
            qk = qrow * 65
            qd = qrow * 33
            zero = jnp.zeros((16,), jnp.float32)
            accs = (zero,) * HD

            def k_body(k, accs):
                base = qk + k
                rowb = plsc.load_gather(idx_v, [base])
                w = plsc.load_gather(cw_v, [base])
                accs = list(accs)
                for j in range(16):
                    g = plsc.load_gather(tbl_v, [rowb + j])
                    gb = plsc.bitcast(g, jnp.bfloat16)
                    ea, eb = plsc.unpack(
                        gb, format=plsc.PackFormat.INTERLEAVED,
                        preferred_element_type=jnp.float32)
                    accs[2 * j] = accs[2 * j] + ea * w
                    accs[2 * j + 1] = accs[2 * j + 1] + eb * w
                return tuple(accs)

            accs = lax.fori_loop(0, NK, k_body, accs)
            for d in range(HD):
                plsc.store_scatter(out_v, [qd + d], accs[d])
            return 0

        lax.fori_loop(0, CH // 16, qblk_body, 0)
        pltpu.sync_copy(out_v, out_hbm.at[pl.ds(wid * (LQ * 33) + q0 * 33, CH * 33)])
        return 0

    lax.fori_loop(0, NCH, chunk_body, 0)


@jax.jit
def _run(query, reference_points, value, W_value, b_value, W_off, b_off,
         W_attn, b_attn, W_out, b_out):
    WvT = W_value.T
    WoxT = W_off[0::2].T
    WoyT = W_off[1::2].T
    box = b_off[0::2]
    boy = b_off[1::2]
    WaT = W_attn.T
    ref_flat = reference_points.reshape(BS, LQ, NL * 2)

    tok = lambda i, b: (b, i, 0)
    full = lambda i, b: (0, 0)
    vec = lambda i, b: (0,)
    v_p, idx_p, cw_p = pl.pallas_call(
        _prep_body,
        grid=(NT, BS),
        in_specs=[
            pl.BlockSpec((1, TQ, D), tok),
            pl.BlockSpec((1, TQ, D), tok),
            pl.BlockSpec((1, TQ, NL * 2), tok),
            pl.BlockSpec((D, D), full),
            pl.BlockSpec((D,), vec),
            pl.BlockSpec((D, 128), full),
            pl.BlockSpec((128,), vec),
            pl.BlockSpec((D, 128), full),
            pl.BlockSpec((128,), vec),
            pl.BlockSpec((D, 128), full),
            pl.BlockSpec((128,), vec),
            pl.BlockSpec((128, 128), full),
            pl.BlockSpec((8, 128), full),
            pl.BlockSpec((8, 128), full),
            pl.BlockSpec((128,), vec),
            pl.BlockSpec((128,), vec),
            pl.BlockSpec((128,), vec),
        ],
        out_specs=[
            pl.BlockSpec((1, TQ, D), tok),
            pl.BlockSpec((1, TQ, 4, 128), lambda i, b: (b, i, 0, 0)),
            pl.BlockSpec((1, TQ, 4, 128), lambda i, b: (b, i, 0, 0)),
        ],
        out_shape=[
            jax.ShapeDtypeStruct((BS, LQ, D), jnp.float32),
            jax.ShapeDtypeStruct((BS, LQ, 4, 128), jnp.int32),
            jax.ShapeDtypeStruct((BS, LQ, 4, 128), jnp.float32),
        ],
    )(value, query, ref_flat, WvT, b_value, WoxT, box, WoyT, boy,
      WaT, b_attn, jnp.asarray(_G), jnp.asarray(_Ex), jnp.asarray(_Ey),
      jnp.asarray(_colW), jnp.asarray(_colH), jnp.asarray(_colS))

    v4 = v_p.reshape(BS, LV, NH, HD).transpose(0, 2, 1, 3)
    vb = v4.astype(jnp.bfloat16).reshape(BS, NH, LV, 16, 2)
    tbl = lax.bitcast_convert_type(vb, jnp.int32)
    tbl = jnp.pad(tbl, ((0, 0), (0, 0), (0, 0), (0, 1)))
    tbl = tbl.reshape(BS * NH * LV * 17)
    idx_r = idx_p.reshape(BS, LQ, 4, NH, 16).transpose(0, 3, 1, 2, 4)
    idx_r = idx_r.reshape(BS, NH, LQ, NK)
    idx_r = jnp.pad(idx_r, ((0, 0), (0, 0), (0, 0), (0, 1)))
    idx_r = idx_r.reshape(BS * NH * LQ * 65)
    cw_r = cw_p.reshape(BS, LQ, 4, NH, 16).transpose(0, 3, 1, 2, 4)
    cw_r = cw_r.reshape(BS, NH, LQ, NK)
    cw_r = jnp.pad(cw_r, ((0, 0), (0, 0), (0, 0), (0, 1)))
    cw_r = cw_r.reshape(BS * NH * LQ * 65)

    samp = _sc_sample(tbl, idx_r, cw_r)
    samp = samp.reshape(BS, NH, LQ, 33)

    out = pl.pallas_call(
        _outproj_body,
        grid=(NT, BS),
        in_specs=[
            pl.BlockSpec((1, NH, TQ, 33), lambda i, b: (b, 0, i, 0)),
            pl.BlockSpec((D, D), full),
            pl.BlockSpec((D,), vec),
        ],
        out_specs=pl.BlockSpec((1, TQ, D), tok),
        out_shape=jax.ShapeDtypeStruct((BS, LQ, D), jnp.float32),
    )(samp, W_out.T, b_out)
    return out


def kernel(query, reference_points, value, value_spatial_shapes,
           value_level_start_index, W_value, b_value, W_off, b_off,
           W_attn, b_attn, W_out, b_out):
    return _run(query, reference_points, value, W_value, b_value,
                W_off, b_off, W_attn, b_attn, W_out, b_out)


# speedup vs baseline: 1.1633x; 1.0418x over previous
import functools
import math

import jax
import jax.numpy as jnp
import numpy as np
from jax import lax
from jax.experimental import pallas as pl
from jax.experimental.pallas import tpu as pltpu
from jax.experimental.pallas import tpu_sc as plsc

SS = ((64, 64), (32, 32), (16, 16), (8, 8))
BS = 4
LQ = 5440
D = 256
NH = 8
NL = 4
NP = 4
HD = D // NH
TP = NH * NL * NP
LV = sum(h * w for h, w in SS)
TQ = 320
NT = LQ // TQ
CH = 160
NCH = LQ // CH
NK = 4 * NL * NP

_colW = np.zeros((128,), np.float32)
_colH = np.zeros((128,), np.float32)
_colS = np.zeros((128,), np.float32)
_starts = [0, 4096, 5120, 5376]
for _h in range(NH):
    for _l in range(NL):
        for _p in range(NP):
            c = _h * 16 + _l * 4 + _p
            _colW[c] = SS[_l][1]
            _colH[c] = SS[_l][0]
            _colS[c] = _starts[_l]
_G = np.zeros((128, 128), np.float32)
for c in range(128):
    g = c // 16
    _G[c, g * 16:(g + 1) * 16] = 1.0
_Ex = np.zeros((8, 128), np.float32)
_Ey = np.zeros((8, 128), np.float32)
for c in range(128):
    _l = (c % 16) // 4
    _Ex[2 * _l, c] = 1.0
    _Ey[2 * _l + 1, c] = 1.0

_CORNERS = ((0.0, 0.0), (0.0, 1.0), (1.0, 0.0), (1.0, 1.0))
_HI = lax.Precision.HIGHEST


def _prep_body(value_b, query_b, ref_b, WvT, bv, WoxT, box, WoyT, boy,
               WaT, ba, G, Ex, Ey, colW, colH, colS,
               v_out, idx_out, cw_out):
    value_b = value_b[0]
    query_b = query_b[0]
    ref_b = ref_b[0]
    v = jnp.dot(value_b, WvT[...], preferred_element_type=jnp.float32,
                precision=lax.Precision.DEFAULT) + bv[...]
    v_out[0] = v
    offx = jnp.dot(query_b, WoxT[...], preferred_element_type=jnp.float32,
                   precision=_HI) + box[...]
    offy = jnp.dot(query_b, WoyT[...], preferred_element_type=jnp.float32,
                   precision=_HI) + boy[...]
    logits = jnp.dot(query_b, WaT[...], preferred_element_type=jnp.float32,
                     precision=_HI) + ba[...]
    m = jnp.max(logits, axis=-1, keepdims=True)
    e = jnp.exp(logits - m)
    denom = jnp.dot(e, G[...], preferred_element_type=jnp.float32,
                    precision=_HI)
    aw = e / denom
    refx = jnp.dot(ref_b, Ex[...], preferred_element_type=jnp.float32,
                   precision=_HI)
    refy = jnp.dot(ref_b, Ey[...], preferred_element_type=jnp.float32,
                   precision=_HI)
    cW = colW[...]
    cH = colH[...]
    cS = colS[...]
    px = refx * cW + offx - 0.5
    py = refy * cH + offy - 0.5
    x0 = jnp.floor(px)
    y0 = jnp.floor(py)
    fx = px - x0
    fy = py - y0
    for ci, (dx, dy) in enumerate(_CORNERS):
        xi = x0 + dx
        yi = y0 + dy
        valid = ((xi >= 0.0) & (xi <= cW - 1.0)
                 & (yi >= 0.0) & (yi <= cH - 1.0))
        wx = fx if dx else 1.0 - fx
        wy = fy if dy else 1.0 - fy
        cw = jnp.where(valid, aw * wx * wy, 0.0)
        xc = jnp.clip(xi, 0.0, cW - 1.0)
        yc = jnp.clip(yi, 0.0, cH - 1.0)
        rows = (cS + yc * cW + xc).astype(jnp.int32)
        idx_out[0, :, ci, :] = rows * 17
        cw_out[0, :, ci, :] = cw


def _outproj_body(x_b, WoT, bo, out_b):
    acc = bo[...]
    for hh in range(NH):
        acc = acc + jnp.dot(x_b[0, hh, :, :HD],
                            WoT[pl.ds(hh * HD, HD), :],
                            preferred_element_type=jnp.float32,
                            precision=_HI)
    out_b[0] = acc


_sc_mesh = plsc.VectorSubcoreMesh(core_axis_name="c", subcore_axis_name="s")


@functools.partial(
    pl.kernel,
    out_type=jax.ShapeDtypeStruct((BS * NH * LQ * 33,), jnp.float32),
    mesh=_sc_mesh,
    compiler_params=pltpu.CompilerParams(needs_layout_passes=False),
    scratch_types=[
        pltpu.VMEM((LV * 17,), jnp.int32),
        pltpu.VMEM((CH * 65,), jnp.int32),
        pltpu.VMEM((CH * 65,), jnp.float32),
        pltpu.VMEM((CH * 33,), jnp.float32),
    ],
)
def _sc_sample(tbl_hbm, idx_hbm, cw_hbm, out_hbm, tbl_v, idx_v, cw_v, out_v):
    cid = lax.axis_index("c")
    sid = lax.axis_index("s")
    wid = sid * 2 + cid
    pltpu.sync_copy(tbl_hbm.at[pl.ds(wid * (LV * 17), LV * 17)], tbl_v)
    lanes = lax.iota(jnp.int32, 16)

    def chunk_body(ci, _):
        q0 = ci * CH
        off = wid * (LQ * 65) + q0 * 65
        pltpu.sync_copy(idx_hbm.at[pl.ds(off, CH * 65)], idx_v)
        pltpu.sync_copy(cw_hbm.at[pl.ds(off, CH * 65)], cw_v)

        def qblk_body(qi, _):
            qrow = qi * 16 + lanes
            qk = qrow * 65
            qd = qrow * 33
            zero = jnp.zeros((16,), jnp.float32)
            accs = (zero,) * HD

            def k_body(k, accs):
                base = qk + k
                rowb = plsc.load_gather(idx_v, [base])
                w = plsc.load_gather(cw_v, [base])
                w2 = plsc.pack(w, w, format=plsc.PackFormat.INTERLEAVED)
                accs = list(accs)
                for j in range(16):
                    g = plsc.load_gather(tbl_v, [rowb + j])
                    gb = plsc.bitcast(g, jnp.bfloat16)
                    ea, eb = plsc.unpack(
                        gb * w2, format=plsc.PackFormat.INTERLEAVED,
                        preferred_element_type=jnp.float32)
                    accs[2 * j] = accs[2 * j] + ea
                    accs[2 * j + 1] = accs[2 * j + 1] + eb
                return tuple(accs)

            accs = lax.fori_loop(0, NK, k_body, accs)
            for d in range(HD):
                plsc.store_scatter(out_v, [qd + d], accs[d])
            return 0

        lax.fori_loop(0, CH // 16, qblk_body, 0)
        pltpu.sync_copy(out_v, out_hbm.at[pl.ds(wid * (LQ * 33) + q0 * 33, CH * 33)])
        return 0

    lax.fori_loop(0, NCH, chunk_body, 0)


@jax.jit
def _run(query, reference_points, value, W_value, b_value, W_off, b_off,
         W_attn, b_attn, W_out, b_out):
    WvT = W_value.T
    WoxT = W_off[0::2].T
    WoyT = W_off[1::2].T
    box = b_off[0::2]
    boy = b_off[1::2]
    WaT = W_attn.T
    ref_flat = reference_points.reshape(BS, LQ, NL * 2)

    tok = lambda i, b: (b, i, 0)
    full = lambda i, b: (0, 0)
    vec = lambda i, b: (0,)
    v_p, idx_p, cw_p = pl.pallas_call(
        _prep_body,
        grid=(NT, BS),
        in_specs=[
            pl.BlockSpec((1, TQ, D), tok),
            pl.BlockSpec((1, TQ, D), tok),
            pl.BlockSpec((1, TQ, NL * 2), tok),
            pl.BlockSpec((D, D), full),
            pl.BlockSpec((D,), vec),
            pl.BlockSpec((D, 128), full),
            pl.BlockSpec((128,), vec),
            pl.BlockSpec((D, 128), full),
            pl.BlockSpec((128,), vec),
            pl.BlockSpec((D, 128), full),
            pl.BlockSpec((128,), vec),
            pl.BlockSpec((128, 128), full),
            pl.BlockSpec((8, 128), full),
            pl.BlockSpec((8, 128), full),
            pl.BlockSpec((128,), vec),
            pl.BlockSpec((128,), vec),
            pl.BlockSpec((128,), vec),
        ],
        out_specs=[
            pl.BlockSpec((1, TQ, D), tok),
            pl.BlockSpec((1, TQ, 4, 128), lambda i, b: (b, i, 0, 0)),
            pl.BlockSpec((1, TQ, 4, 128), lambda i, b: (b, i, 0, 0)),
        ],
        out_shape=[
            jax.ShapeDtypeStruct((BS, LQ, D), jnp.float32),
            jax.ShapeDtypeStruct((BS, LQ, 4, 128), jnp.int32),
            jax.ShapeDtypeStruct((BS, LQ, 4, 128), jnp.float32),
        ],
    )(value, query, ref_flat, WvT, b_value, WoxT, box, WoyT, boy,
      WaT, b_attn, jnp.asarray(_G), jnp.asarray(_Ex), jnp.asarray(_Ey),
      jnp.asarray(_colW), jnp.asarray(_colH), jnp.asarray(_colS))

    v4 = v_p.reshape(BS, LV, NH, HD).transpose(0, 2, 1, 3)
    vb = v4.astype(jnp.bfloat16).reshape(BS, NH, LV, 16, 2)
    tbl = lax.bitcast_convert_type(vb, jnp.int32)
    tbl = jnp.pad(tbl, ((0, 0), (0, 0), (0, 0), (0, 1)))
    tbl = tbl.reshape(BS * NH * LV * 17)
    idx_r = idx_p.reshape(BS, LQ, 4, NH, 16).transpose(0, 3, 1, 2, 4)
    idx_r = idx_r.reshape(BS, NH, LQ, NK)
    idx_r = jnp.pad(idx_r, ((0, 0), (0, 0), (0, 0), (0, 1)))
    idx_r = idx_r.reshape(BS * NH * LQ * 65)
    cw_r = cw_p.reshape(BS, LQ, 4, NH, 16).transpose(0, 3, 1, 2, 4)
    cw_r = cw_r.reshape(BS, NH, LQ, NK)
    cw_r = jnp.pad(cw_r, ((0, 0), (0, 0), (0, 0), (0, 1)))
    cw_r = cw_r.reshape(BS * NH * LQ * 65)

    samp = _sc_sample(tbl, idx_r, cw_r)
    samp = samp.reshape(BS, NH, LQ, 33)

    out = pl.pallas_call(
        _outproj_body,
        grid=(NT, BS),
        in_specs=[
            pl.BlockSpec((1, NH, TQ, 33), lambda i, b: (b, 0, i, 0)),
            pl.BlockSpec((D, D), full),
            pl.BlockSpec((D,), vec),
        ],
        out_specs=pl.BlockSpec((1, TQ, D), tok),
        out_shape=jax.ShapeDtypeStruct((BS, LQ, D), jnp.float32),
    )(samp, W_out.T, b_out)
    return out


def kernel(query, reference_points, value, value_spatial_shapes,
           value_level_start_index, W_value, b_value, W_off, b_off,
           W_attn, b_attn, W_out, b_out):
    return _run(query, reference_points, value, W_value, b_value,
                W_off, b_off, W_attn, b_attn, W_out, b_out)
